# baseline (speedup 1.0000x reference)
import functools

import jax
import jax.numpy as jnp
from jax import lax
from jax.experimental import pallas as pl
from jax.experimental.pallas import tpu as pltpu
from jax.experimental.pallas import tpu_sc as plsc

N = 10000
E = 320000
H = 128
NG = 64
NC = 2
NS = 16
NW = NC * NS
EPW = E // NW
CH = 128
NP = 10112
RPT = NP // NS
NCK = E // CH
CPW = NCK // NW
XW = NCK - CPW * NW
NBUF = 6
ROUNDS = CPW // NBUF
NBUF_S = 3
ROUNDS_S = CPW // NBUF_S


def _mesh():
    return plsc.VectorSubcoreMesh(
        core_axis_name="c", subcore_axis_name="s", num_cores=NC, num_subcores=NS
    )


def _sc_gather(pn, idx):
    @functools.partial(
        pl.kernel,
        out_type=jax.ShapeDtypeStruct((E, H), jnp.float32),
        mesh=_mesh(),
        scratch_types=[
            [pltpu.VMEM((CH,), jnp.int32) for _ in range(NBUF)],
            [pltpu.VMEM((CH, H), jnp.float32) for _ in range(NBUF)],
            [pltpu.SemaphoreType.DMA for _ in range(NBUF)],
            [pltpu.SemaphoreType.DMA for _ in range(NBUF)],
            [pltpu.SemaphoreType.DMA for _ in range(NBUF)],
        ],
    )
    def k(pn_hbm, idx_hbm, out_hbm, idxs, rows, isems, gsems, ssems):
        wid = lax.axis_index("s") * NC + lax.axis_index("c")
        base = wid * CPW

        def rnd(g, carry):
            ih = []
            for b in range(NBUF):
                o = pl.multiple_of((base + g * NBUF + b) * CH, 8)
                ih.append(pltpu.async_copy(idx_hbm.at[pl.ds(o, CH)], idxs[b], isems[b]))
            gh = []
            for b in range(NBUF):
                ih[b].wait()
                gh.append(pltpu.async_copy(pn_hbm.at[idxs[b]], rows[b], gsems[b]))
            sh = []
            for b in range(NBUF):
                o = pl.multiple_of((base + g * NBUF + b) * CH, 8)
                gh[b].wait()
                sh.append(pltpu.async_copy(rows[b], out_hbm.at[pl.ds(o, CH)], ssems[b]))
            for b in range(NBUF):
                sh[b].wait()
            return carry

        lax.fori_loop(0, ROUNDS, rnd, 0)

        @pl.when(wid < XW)
        def _extra():
            o = pl.multiple_of((NW * CPW + wid) * CH, 8)
            pltpu.sync_copy(idx_hbm.at[pl.ds(o, CH)], idxs[0])
            pltpu.async_copy(pn_hbm.at[idxs[0]], rows[0], gsems[0]).wait()
            pltpu.sync_copy(rows[0], out_hbm.at[pl.ds(o, CH)])

    return k(pn, idx)


def _sc_scatter(ue, ridx, zrows):
    @functools.partial(
        pl.kernel,
        out_type=jax.ShapeDtypeStruct((NC * NP, H), jnp.float32),
        mesh=_mesh(),
        scratch_types=[
            [pltpu.VMEM((CH,), jnp.int32) for _ in range(NBUF_S)],
            [pltpu.VMEM((CH, H), jnp.float32) for _ in range(NBUF_S)],
            pltpu.VMEM_SHARED((NP, H), jnp.float32),
            [pltpu.SemaphoreType.DMA for _ in range(NBUF_S)],
            [pltpu.SemaphoreType.DMA for _ in range(NBUF_S)],
            [pltpu.SemaphoreType.DMA for _ in range(NBUF_S)],
        ],
    )
    def k(ue_hbm, ridx_hbm, z_hbm, out_hbm, idxs, bufs, agg_sh, isems, dsems, asems):
        cid = lax.axis_index("c")
        sid = lax.axis_index("s")
        wid = sid * NC + cid
        base = wid * CPW
        r0 = sid * RPT
        pltpu.sync_copy(z_hbm.at[pl.ds(r0, RPT)], agg_sh.at[pl.ds(r0, RPT)])
        plsc.subcore_barrier()

        def rnd(g, carry):
            ih, dh = [], []
            for b in range(NBUF_S):
                o = pl.multiple_of((base + g * NBUF_S + b) * CH, 8)
                ih.append(pltpu.async_copy(ridx_hbm.at[pl.ds(o, CH)], idxs[b], isems[b]))
                dh.append(pltpu.async_copy(ue_hbm.at[pl.ds(o, CH)], bufs[b], dsems[b]))
            ah = []
            for b in range(NBUF_S):
                ih[b].wait()
                dh[b].wait()
                ah.append(pltpu.async_copy(bufs[b], agg_sh.at[idxs[b]], asems[b], add=True))
            for b in range(NBUF_S):
                ah[b].wait()
            return carry

        lax.fori_loop(0, ROUNDS_S, rnd, 0)

        @pl.when(wid < XW)
        def _extra():
            o = pl.multiple_of((NW * CPW + wid) * CH, 8)
            pltpu.sync_copy(ridx_hbm.at[pl.ds(o, CH)], idxs[0])
            pltpu.sync_copy(ue_hbm.at[pl.ds(o, CH)], bufs[0])
            pltpu.sync_copy(bufs[0], agg_sh.at[idxs[0]], add=True)

        plsc.subcore_barrier()
        pltpu.sync_copy(
            agg_sh.at[pl.ds(r0, RPT)], out_hbm.at[pl.ds(cid * NP + r0, RPT)]
        )

    return k(ue, ridx, zrows)


def _tc_matmul(x, wT):
    R = 2000
    K = x.shape[1]

    def body(x_ref, w_ref, o_ref):
        o_ref[...] = jnp.dot(x_ref[...], w_ref[...], preferred_element_type=jnp.float32)

    return pl.pallas_call(
        body,
        grid=(N // R,),
        in_specs=[
            pl.BlockSpec((R, K), lambda i: (i, 0)),
            pl.BlockSpec((K, H), lambda i: (0, 0)),
        ],
        out_specs=pl.BlockSpec((R, H), lambda i: (i, 0)),
        out_shape=jax.ShapeDtypeStruct((N, H), jnp.float32),
    )(x, wT)


def _tc_edge(x, wT, sg, a):
    R = 3200
    K = x.shape[1]

    def body(x_ref, w_ref, sg_ref, a_ref, o_ref):
        acc = jnp.dot(x_ref[...], w_ref[...], preferred_element_type=jnp.float32)
        acc = acc + sg_ref[...]
        o_ref[...] = jnp.where(acc >= 0, acc, acc * a_ref[...])

    return pl.pallas_call(
        body,
        grid=(E // R,),
        in_specs=[
            pl.BlockSpec((R, K), lambda i: (i, 0)),
            pl.BlockSpec((K, H), lambda i: (0, 0)),
            pl.BlockSpec((R, H), lambda i: (i, 0)),
            pl.BlockSpec((1, H), lambda i: (0, 0)),
        ],
        out_specs=pl.BlockSpec((R, H), lambda i: (i, 0)),
        out_shape=jax.ShapeDtypeStruct((E, H), jnp.float32),
    )(x, wT, sg, a)


def _tc_edge_mm(x, wT):
    R = 3200
    K = x.shape[1]

    def body(x_ref, w_ref, o_ref):
        o_ref[...] = jnp.dot(x_ref[...], w_ref[...], preferred_element_type=jnp.float32)

    return pl.pallas_call(
        body,
        grid=(E // R,),
        in_specs=[
            pl.BlockSpec((R, K), lambda i: (i, 0)),
            pl.BlockSpec((K, H), lambda i: (0, 0)),
        ],
        out_specs=pl.BlockSpec((R, H), lambda i: (i, 0)),
        out_shape=jax.ShapeDtypeStruct((E, H), jnp.float32),
    )(x, wT)


def _tc_edge_act(pe, sg, a):
    R = 3200

    def body(pe_ref, sg_ref, a_ref, o_ref):
        acc = pe_ref[...] + sg_ref[...]
        o_ref[...] = jnp.where(acc >= 0, acc, acc * a_ref[...])

    return pl.pallas_call(
        body,
        grid=(E // R,),
        in_specs=[
            pl.BlockSpec((R, H), lambda i: (i, 0)),
            pl.BlockSpec((R, H), lambda i: (i, 0)),
            pl.BlockSpec((1, H), lambda i: (0, 0)),
        ],
        out_specs=pl.BlockSpec((R, H), lambda i: (i, 0)),
        out_shape=jax.ShapeDtypeStruct((E, H), jnp.float32),
    )(pe, sg, a)


def _tc_node(agg0, agg1, x, nwaT, nwnT, na, ewnT):
    R = 2000
    K = x.shape[1]

    def body(a0, a1, x_ref, wa, wn, na_ref, ewn, n_out, pn_out):
        h = jnp.dot(a0[...] + a1[...], wa[...], preferred_element_type=jnp.float32)
        h = h + jnp.dot(x_ref[...], wn[...], preferred_element_type=jnp.float32)
        nn = jnp.where(h >= 0, h, h * na_ref[...])
        n_out[...] = nn
        pn_out[...] = jnp.dot(nn, ewn[...], preferred_element_type=jnp.float32)

    return pl.pallas_call(
        body,
        grid=(N // R,),
        in_specs=[
            pl.BlockSpec((R, H), lambda i: (i, 0)),
            pl.BlockSpec((R, H), lambda i: (i, 0)),
            pl.BlockSpec((R, K), lambda i: (i, 0)),
            pl.BlockSpec((H, H), lambda i: (0, 0)),
            pl.BlockSpec((K, H), lambda i: (0, 0)),
            pl.BlockSpec((1, H), lambda i: (0, 0)),
            pl.BlockSpec((H, H), lambda i: (0, 0)),
        ],
        out_specs=[
            pl.BlockSpec((R, H), lambda i: (i, 0)),
            pl.BlockSpec((R, H), lambda i: (i, 0)),
        ],
        out_shape=[
            jax.ShapeDtypeStruct((N, H), jnp.float32),
            jax.ShapeDtypeStruct((N, H), jnp.float32),
        ],
    )(agg0, agg1, x, nwaT, nwnT, na, ewnT)


def _tc_final(agg0, agg1, x, nwaT, nwnT, na, batch3, fcw, fcb, fca, owT, ob):
    R = 2000
    G = N // R

    def body(a0, a1, x_ref, wa, wn, na_ref, b_ref,
             w1, b1, p1, w2, b2, p2, w3, b3, p3, ow, ob_ref, out_ref, acc):
        j = pl.program_id(0)

        @pl.when(j == 0)
        def _init():
            acc[...] = jnp.zeros_like(acc)

        h = jnp.dot(a0[...] + a1[...], wa[...], preferred_element_type=jnp.float32)
        h = h + jnp.dot(x_ref[...], wn[...], preferred_element_type=jnp.float32)
        nn = jnp.where(h >= 0, h, h * na_ref[...])
        seg = b_ref[0]
        ohT = (lax.broadcasted_iota(jnp.int32, (NG, R), 0) == seg).astype(jnp.float32)
        acc[...] += jnp.dot(ohT, nn, preferred_element_type=jnp.float32)

        @pl.when(j == G - 1)
        def _head():
            u = acc[...]
            for w, b, p in ((w1, b1, p1), (w2, b2, p2), (w3, b3, p3)):
                v = jnp.dot(u, w[...], preferred_element_type=jnp.float32) + b[...]
                u = jnp.where(v >= 0, v, v * p[...])
            out_ref[...] = jnp.dot(u, ow[...], preferred_element_type=jnp.float32) + ob_ref[...]

    wspec = pl.BlockSpec((H, H), lambda i: (0, 0))
    vspec = pl.BlockSpec((1, H), lambda i: (0, 0))
    return pl.pallas_call(
        body,
        grid=(G,),
        in_specs=[
            pl.BlockSpec((R, H), lambda i: (i, 0)),
            pl.BlockSpec((R, H), lambda i: (i, 0)),
            pl.BlockSpec((R, H), lambda i: (i, 0)),
            wspec,
            wspec,
            vspec,
            pl.BlockSpec((1, 1, R), lambda i: (i, 0, 0)),
            wspec, vspec, vspec,
            wspec, vspec, vspec,
            wspec, vspec, vspec,
            pl.BlockSpec((H, 1), lambda i: (0, 0)),
            pl.BlockSpec((1, 1), lambda i: (0, 0)),
        ],
        out_specs=pl.BlockSpec((NG, 1), lambda i: (0, 0)),
        out_shape=jax.ShapeDtypeStruct((NG, 1), jnp.float32),
        scratch_shapes=[pltpu.VMEM((NG, H), jnp.float32)],
    )(agg0, agg1, x, nwaT, nwnT, na, batch3,
      fcw[0], fcb[0], fca[0], fcw[1], fcb[1], fca[1], fcw[2], fcb[2], fca[2],
      owT, ob)


def kernel(n, e, e_i, batch, params):
    recv = e_i[0]
    send = e_i[1]
    gl = params["gl"]
    fc = params["fc"]

    nd0 = n.shape[1]
    n48 = jnp.pad(n, ((0, 0), (0, 48 - nd0)))
    e16 = jnp.pad(e, ((0, 0), (0, 16 - e.shape[1])))
    ewn0T = jnp.pad(gl[0]["ew"][:, :nd0].T, ((0, 48 - nd0), (0, 0)))
    ewe0T = jnp.pad(gl[0]["ew"][:, nd0:].T, ((0, 16 - e.shape[1]), (0, 0)))
    nwa0T = gl[0]["nw"][:, :H].T
    nwn0T = jnp.pad(gl[0]["nw"][:, H:].T, ((0, 48 - nd0), (0, 0)))
    ewn1T, ewe1T = gl[1]["ew"][:, :H].T, gl[1]["ew"][:, H:].T
    nwa1T, nwn1T = gl[1]["nw"][:, :H].T, gl[1]["nw"][:, H:].T
    ewn2T, ewe2T = gl[2]["ew"][:, :H].T, gl[2]["ew"][:, H:].T
    nwa2T, nwn2T = gl[2]["nw"][:, :H].T, gl[2]["nw"][:, H:].T
    ea = [lp["ea"].reshape(1, H) for lp in gl]
    na = [lp["na"].reshape(1, H) for lp in gl]
    fcw = [lp["w"].T for lp in fc]
    fcb = [lp["b"].reshape(1, H) for lp in fc]
    fca = [lp["a"].reshape(1, H) for lp in fc]
    owT = params["ow"].T
    ob = params["ob"].reshape(1, 1)
    batch3 = batch.reshape(N // 2000, 1, 2000)
    zrows = jnp.zeros((NP, H), jnp.float32)

    pn0 = _tc_matmul(n48, ewn0T)
    sg0 = _sc_gather(pn0, send)
    ue0 = _tc_edge(e16, ewe0T, sg0, ea[0])
    ag0 = _sc_scatter(ue0, recv, zrows)
    n1, pn1 = _tc_node(ag0[:N], ag0[NP:NP + N], n48, nwa0T, nwn0T, na[0], ewn1T)
    sg1 = _sc_gather(pn1, send)
    pe1 = _tc_edge_mm(ue0, ewe1T)
    ue1 = _tc_edge_act(pe1, sg1, ea[1])
    ag1 = _sc_scatter(ue1, recv, zrows)
    n2, pn2 = _tc_node(ag1[:N], ag1[NP:NP + N], n1, nwa1T, nwn1T, na[1], ewn2T)
    sg2 = _sc_gather(pn2, send)
    pe2 = _tc_edge_mm(ue1, ewe2T)
    ue2 = _tc_edge_act(pe2, sg2, ea[2])
    ag2 = _sc_scatter(ue2, recv, zrows)
    return _tc_final(ag2[:N], ag2[NP:NP + N], n2, nwa2T, nwn2T, na[2], batch3,
                     fcw, fcb, fca, owT, ob)


# speedup vs baseline: 1.1862x; 1.1862x over previous
import functools

import jax
import jax.numpy as jnp
from jax import lax
from jax.experimental import pallas as pl
from jax.experimental.pallas import tpu as pltpu
from jax.experimental.pallas import tpu_sc as plsc

N = 10000
E = 320000
H = 128
NG = 64
NC = 2
NS = 16
NW = NC * NS
EPW = E // NW
CH = 128
NP = 10112
RPT = NP // NS
NCK = E // CH
CPW = NCK // NW
XW = NCK - CPW * NW
NBUF = 6
ROUNDS = CPW // NBUF
NBUF_S = 3
ROUNDS_S = CPW // NBUF_S


def _mesh():
    return plsc.VectorSubcoreMesh(
        core_axis_name="c", subcore_axis_name="s", num_cores=NC, num_subcores=NS
    )


def _sc_gather(pn, idx):
    @functools.partial(
        pl.kernel,
        out_type=jax.ShapeDtypeStruct((E, H), jnp.float32),
        mesh=_mesh(),
        scratch_types=[
            [pltpu.VMEM((CH,), jnp.int32) for _ in range(NBUF)],
            [pltpu.VMEM((CH, H), jnp.float32) for _ in range(NBUF)],
            [pltpu.SemaphoreType.DMA for _ in range(NBUF)],
            [pltpu.SemaphoreType.DMA for _ in range(NBUF)],
            [pltpu.SemaphoreType.DMA for _ in range(NBUF)],
        ],
    )
    def k(pn_hbm, idx_hbm, out_hbm, idxs, rows, isems, gsems, ssems):
        wid = lax.axis_index("s") * NC + lax.axis_index("c")
        base = wid * CPW

        def rnd(g, carry):
            ih = []
            for b in range(NBUF):
                o = pl.multiple_of((base + g * NBUF + b) * CH, 8)

                @pl.when(g > 0)
                def _drain(b=b):
                    pltpu.make_async_copy(
                        rows[b], out_hbm.at[pl.ds(0, CH)], ssems[b]
                    ).wait()

                ih.append(pltpu.async_copy(idx_hbm.at[pl.ds(o, CH)], idxs[b], isems[b]))
            gh = []
            for b in range(NBUF):
                ih[b].wait()
                gh.append(pltpu.async_copy(pn_hbm.at[idxs[b]], rows[b], gsems[b]))
            for b in range(NBUF):
                o = pl.multiple_of((base + g * NBUF + b) * CH, 8)
                gh[b].wait()
                pltpu.async_copy(rows[b], out_hbm.at[pl.ds(o, CH)], ssems[b])
            return carry

        lax.fori_loop(0, ROUNDS, rnd, 0)
        for b in range(NBUF):
            pltpu.make_async_copy(rows[b], out_hbm.at[pl.ds(0, CH)], ssems[b]).wait()

        @pl.when(wid < XW)
        def _extra():
            o = pl.multiple_of((NW * CPW + wid) * CH, 8)
            pltpu.sync_copy(idx_hbm.at[pl.ds(o, CH)], idxs[0])
            pltpu.async_copy(pn_hbm.at[idxs[0]], rows[0], gsems[0]).wait()
            pltpu.sync_copy(rows[0], out_hbm.at[pl.ds(o, CH)])

    return k(pn, idx)


def _sc_scatter(ue, ridx, zrows):
    @functools.partial(
        pl.kernel,
        out_type=jax.ShapeDtypeStruct((NC * NP, H), jnp.float32),
        mesh=_mesh(),
        scratch_types=[
            [pltpu.VMEM((CH,), jnp.int32) for _ in range(NBUF_S)],
            [pltpu.VMEM((CH, H), jnp.float32) for _ in range(NBUF_S)],
            pltpu.VMEM_SHARED((NP, H), jnp.float32),
            [pltpu.SemaphoreType.DMA for _ in range(NBUF_S)],
            [pltpu.SemaphoreType.DMA for _ in range(NBUF_S)],
            [pltpu.SemaphoreType.DMA for _ in range(NBUF_S)],
        ],
    )
    def k(ue_hbm, ridx_hbm, z_hbm, out_hbm, idxs, bufs, agg_sh, isems, dsems, asems):
        cid = lax.axis_index("c")
        sid = lax.axis_index("s")
        wid = sid * NC + cid
        base = wid * CPW
        r0 = sid * RPT
        pltpu.sync_copy(z_hbm.at[pl.ds(r0, RPT)], agg_sh.at[pl.ds(r0, RPT)])
        plsc.subcore_barrier()

        def rnd(g, carry):
            ih, dh = [], []
            for b in range(NBUF_S):
                o = pl.multiple_of((base + g * NBUF_S + b) * CH, 8)

                @pl.when(g > 0)
                def _drain(b=b):
                    pltpu.make_async_copy(bufs[b], agg_sh.at[idxs[b]], asems[b]).wait()

                ih.append(pltpu.async_copy(ridx_hbm.at[pl.ds(o, CH)], idxs[b], isems[b]))
                dh.append(pltpu.async_copy(ue_hbm.at[pl.ds(o, CH)], bufs[b], dsems[b]))
            for b in range(NBUF_S):
                ih[b].wait()
                dh[b].wait()
                pltpu.async_copy(bufs[b], agg_sh.at[idxs[b]], asems[b], add=True)
            return carry

        lax.fori_loop(0, ROUNDS_S, rnd, 0)
        for b in range(NBUF_S):
            pltpu.make_async_copy(bufs[b], agg_sh.at[idxs[b]], asems[b]).wait()

        @pl.when(wid < XW)
        def _extra():
            o = pl.multiple_of((NW * CPW + wid) * CH, 8)
            pltpu.sync_copy(ridx_hbm.at[pl.ds(o, CH)], idxs[0])
            pltpu.sync_copy(ue_hbm.at[pl.ds(o, CH)], bufs[0])
            pltpu.sync_copy(bufs[0], agg_sh.at[idxs[0]], add=True)

        plsc.subcore_barrier()
        pltpu.sync_copy(
            agg_sh.at[pl.ds(r0, RPT)], out_hbm.at[pl.ds(cid * NP + r0, RPT)]
        )

    return k(ue, ridx, zrows)


def _tc_matmul(x, wT):
    R = 2000
    K = x.shape[1]

    def body(x_ref, w_ref, o_ref):
        o_ref[...] = jnp.dot(x_ref[...], w_ref[...], preferred_element_type=jnp.float32)

    return pl.pallas_call(
        body,
        grid=(N // R,),
        in_specs=[
            pl.BlockSpec((R, K), lambda i: (i, 0)),
            pl.BlockSpec((K, H), lambda i: (0, 0)),
        ],
        out_specs=pl.BlockSpec((R, H), lambda i: (i, 0)),
        out_shape=jax.ShapeDtypeStruct((N, H), jnp.float32),
    )(x, wT)


def _tc_edge(x, wT, sg, a):
    R = 3200
    K = x.shape[1]

    def body(x_ref, w_ref, sg_ref, a_ref, o_ref):
        acc = jnp.dot(x_ref[...], w_ref[...], preferred_element_type=jnp.float32)
        acc = acc + sg_ref[...]
        o_ref[...] = jnp.where(acc >= 0, acc, acc * a_ref[...])

    return pl.pallas_call(
        body,
        grid=(E // R,),
        in_specs=[
            pl.BlockSpec((R, K), lambda i: (i, 0)),
            pl.BlockSpec((K, H), lambda i: (0, 0)),
            pl.BlockSpec((R, H), lambda i: (i, 0)),
            pl.BlockSpec((1, H), lambda i: (0, 0)),
        ],
        out_specs=pl.BlockSpec((R, H), lambda i: (i, 0)),
        out_shape=jax.ShapeDtypeStruct((E, H), jnp.float32),
    )(x, wT, sg, a)


def _tc_edge_mm(x, wT):
    R = 3200
    K = x.shape[1]

    def body(x_ref, w_ref, o_ref):
        o_ref[...] = jnp.dot(x_ref[...], w_ref[...], preferred_element_type=jnp.float32)

    return pl.pallas_call(
        body,
        grid=(E // R,),
        in_specs=[
            pl.BlockSpec((R, K), lambda i: (i, 0)),
            pl.BlockSpec((K, H), lambda i: (0, 0)),
        ],
        out_specs=pl.BlockSpec((R, H), lambda i: (i, 0)),
        out_shape=jax.ShapeDtypeStruct((E, H), jnp.float32),
    )(x, wT)


def _tc_edge_act(pe, sg, a):
    R = 3200

    def body(pe_ref, sg_ref, a_ref, o_ref):
        acc = pe_ref[...] + sg_ref[...]
        o_ref[...] = jnp.where(acc >= 0, acc, acc * a_ref[...])

    return pl.pallas_call(
        body,
        grid=(E // R,),
        in_specs=[
            pl.BlockSpec((R, H), lambda i: (i, 0)),
            pl.BlockSpec((R, H), lambda i: (i, 0)),
            pl.BlockSpec((1, H), lambda i: (0, 0)),
        ],
        out_specs=pl.BlockSpec((R, H), lambda i: (i, 0)),
        out_shape=jax.ShapeDtypeStruct((E, H), jnp.float32),
    )(pe, sg, a)


def _tc_node(agg0, agg1, x, nwaT, nwnT, na, ewnT):
    R = 2000
    K = x.shape[1]

    def body(a0, a1, x_ref, wa, wn, na_ref, ewn, n_out, pn_out):
        h = jnp.dot(a0[...] + a1[...], wa[...], preferred_element_type=jnp.float32)
        h = h + jnp.dot(x_ref[...], wn[...], preferred_element_type=jnp.float32)
        nn = jnp.where(h >= 0, h, h * na_ref[...])
        n_out[...] = nn
        pn_out[...] = jnp.dot(nn, ewn[...], preferred_element_type=jnp.float32)

    return pl.pallas_call(
        body,
        grid=(N // R,),
        in_specs=[
            pl.BlockSpec((R, H), lambda i: (i, 0)),
            pl.BlockSpec((R, H), lambda i: (i, 0)),
            pl.BlockSpec((R, K), lambda i: (i, 0)),
            pl.BlockSpec((H, H), lambda i: (0, 0)),
            pl.BlockSpec((K, H), lambda i: (0, 0)),
            pl.BlockSpec((1, H), lambda i: (0, 0)),
            pl.BlockSpec((H, H), lambda i: (0, 0)),
        ],
        out_specs=[
            pl.BlockSpec((R, H), lambda i: (i, 0)),
            pl.BlockSpec((R, H), lambda i: (i, 0)),
        ],
        out_shape=[
            jax.ShapeDtypeStruct((N, H), jnp.float32),
            jax.ShapeDtypeStruct((N, H), jnp.float32),
        ],
    )(agg0, agg1, x, nwaT, nwnT, na, ewnT)


def _tc_final(agg0, agg1, x, nwaT, nwnT, na, batch3, fcw, fcb, fca, owT, ob):
    R = 2000
    G = N // R

    def body(a0, a1, x_ref, wa, wn, na_ref, b_ref,
             w1, b1, p1, w2, b2, p2, w3, b3, p3, ow, ob_ref, out_ref, acc):
        j = pl.program_id(0)

        @pl.when(j == 0)
        def _init():
            acc[...] = jnp.zeros_like(acc)

        h = jnp.dot(a0[...] + a1[...], wa[...], preferred_element_type=jnp.float32)
        h = h + jnp.dot(x_ref[...], wn[...], preferred_element_type=jnp.float32)
        nn = jnp.where(h >= 0, h, h * na_ref[...])
        seg = b_ref[0]
        ohT = (lax.broadcasted_iota(jnp.int32, (NG, R), 0) == seg).astype(jnp.float32)
        acc[...] += jnp.dot(ohT, nn, preferred_element_type=jnp.float32)

        @pl.when(j == G - 1)
        def _head():
            u = acc[...]
            for w, b, p in ((w1, b1, p1), (w2, b2, p2), (w3, b3, p3)):
                v = jnp.dot(u, w[...], preferred_element_type=jnp.float32) + b[...]
                u = jnp.where(v >= 0, v, v * p[...])
            out_ref[...] = jnp.dot(u, ow[...], preferred_element_type=jnp.float32) + ob_ref[...]

    wspec = pl.BlockSpec((H, H), lambda i: (0, 0))
    vspec = pl.BlockSpec((1, H), lambda i: (0, 0))
    return pl.pallas_call(
        body,
        grid=(G,),
        in_specs=[
            pl.BlockSpec((R, H), lambda i: (i, 0)),
            pl.BlockSpec((R, H), lambda i: (i, 0)),
            pl.BlockSpec((R, H), lambda i: (i, 0)),
            wspec,
            wspec,
            vspec,
            pl.BlockSpec((1, 1, R), lambda i: (i, 0, 0)),
            wspec, vspec, vspec,
            wspec, vspec, vspec,
            wspec, vspec, vspec,
            pl.BlockSpec((H, 1), lambda i: (0, 0)),
            pl.BlockSpec((1, 1), lambda i: (0, 0)),
        ],
        out_specs=pl.BlockSpec((NG, 1), lambda i: (0, 0)),
        out_shape=jax.ShapeDtypeStruct((NG, 1), jnp.float32),
        scratch_shapes=[pltpu.VMEM((NG, H), jnp.float32)],
    )(agg0, agg1, x, nwaT, nwnT, na, batch3,
      fcw[0], fcb[0], fca[0], fcw[1], fcb[1], fca[1], fcw[2], fcb[2], fca[2],
      owT, ob)


def kernel(n, e, e_i, batch, params):
    recv = e_i[0]
    send = e_i[1]
    gl = params["gl"]
    fc = params["fc"]

    nd0 = n.shape[1]
    n48 = jnp.pad(n, ((0, 0), (0, 48 - nd0)))
    e16 = jnp.pad(e, ((0, 0), (0, 16 - e.shape[1])))
    ewn0T = jnp.pad(gl[0]["ew"][:, :nd0].T, ((0, 48 - nd0), (0, 0)))
    ewe0T = jnp.pad(gl[0]["ew"][:, nd0:].T, ((0, 16 - e.shape[1]), (0, 0)))
    nwa0T = gl[0]["nw"][:, :H].T
    nwn0T = jnp.pad(gl[0]["nw"][:, H:].T, ((0, 48 - nd0), (0, 0)))
    ewn1T, ewe1T = gl[1]["ew"][:, :H].T, gl[1]["ew"][:, H:].T
    nwa1T, nwn1T = gl[1]["nw"][:, :H].T, gl[1]["nw"][:, H:].T
    ewn2T, ewe2T = gl[2]["ew"][:, :H].T, gl[2]["ew"][:, H:].T
    nwa2T, nwn2T = gl[2]["nw"][:, :H].T, gl[2]["nw"][:, H:].T
    ea = [lp["ea"].reshape(1, H) for lp in gl]
    na = [lp["na"].reshape(1, H) for lp in gl]
    fcw = [lp["w"].T for lp in fc]
    fcb = [lp["b"].reshape(1, H) for lp in fc]
    fca = [lp["a"].reshape(1, H) for lp in fc]
    owT = params["ow"].T
    ob = params["ob"].reshape(1, 1)
    batch3 = batch.reshape(N // 2000, 1, 2000)
    zrows = jnp.zeros((NP, H), jnp.float32)

    pn0 = _tc_matmul(n48, ewn0T)
    sg0 = _sc_gather(pn0, send)
    ue0 = _tc_edge(e16, ewe0T, sg0, ea[0])
    ag0 = _sc_scatter(ue0, recv, zrows)
    n1, pn1 = _tc_node(ag0[:N], ag0[NP:NP + N], n48, nwa0T, nwn0T, na[0], ewn1T)
    sg1 = _sc_gather(pn1, send)
    ue1 = _tc_edge(ue0, ewe1T, sg1, ea[1])
    ag1 = _sc_scatter(ue1, recv, zrows)
    n2, pn2 = _tc_node(ag1[:N], ag1[NP:NP + N], n1, nwa1T, nwn1T, na[1], ewn2T)
    sg2 = _sc_gather(pn2, send)
    ue2 = _tc_edge(ue1, ewe2T, sg2, ea[2])
    ag2 = _sc_scatter(ue2, recv, zrows)
    return _tc_final(ag2[:N], ag2[NP:NP + N], n2, nwa2T, nwn2T, na[2], batch3,
                     fcw, fcb, fca, owT, ob)


# speedup vs baseline: 1.2109x; 1.0208x over previous
import functools

import jax
import jax.numpy as jnp
from jax import lax
from jax.experimental import pallas as pl
from jax.experimental.pallas import tpu as pltpu
from jax.experimental.pallas import tpu_sc as plsc

N = 10000
E = 320000
H = 128
NG = 64
NC = 2
NS = 16
NW = NC * NS
EPW = E // NW
CH = 128
NP = 10112
RPT = NP // NS
NCK = E // CH
CPW = NCK // NW
XW = NCK - CPW * NW
NBUF = 6
ROUNDS = CPW // NBUF
NBUF_S = 3
ROUNDS_S = CPW // NBUF_S


def _mesh():
    return plsc.VectorSubcoreMesh(
        core_axis_name="c", subcore_axis_name="s", num_cores=NC, num_subcores=NS
    )


def _sc_gather(pn, idx):
    @functools.partial(
        pl.kernel,
        out_type=jax.ShapeDtypeStruct((E, H), jnp.float32),
        mesh=_mesh(),
        scratch_types=[
            [pltpu.VMEM((CH,), jnp.int32) for _ in range(NBUF)],
            [pltpu.VMEM((CH, H), jnp.float32) for _ in range(NBUF)],
            [pltpu.SemaphoreType.DMA for _ in range(NBUF)],
            [pltpu.SemaphoreType.DMA for _ in range(NBUF)],
            [pltpu.SemaphoreType.DMA for _ in range(NBUF)],
        ],
    )
    def k(pn_hbm, idx_hbm, out_hbm, idxs, rows, isems, gsems, ssems):
        wid = lax.axis_index("s") * NC + lax.axis_index("c")
        base = wid * CPW

        def rnd(g, carry):
            ih = []
            for b in range(NBUF):
                o = pl.multiple_of((base + g * NBUF + b) * CH, 8)

                @pl.when(g > 0)
                def _drain(b=b):
                    pltpu.make_async_copy(
                        rows[b], out_hbm.at[pl.ds(0, CH)], ssems[b]
                    ).wait()

                ih.append(pltpu.async_copy(idx_hbm.at[pl.ds(o, CH)], idxs[b], isems[b]))
            gh = []
            for b in range(NBUF):
                ih[b].wait()
                gh.append(pltpu.async_copy(pn_hbm.at[idxs[b]], rows[b], gsems[b]))
            for b in range(NBUF):
                o = pl.multiple_of((base + g * NBUF + b) * CH, 8)
                gh[b].wait()
                pltpu.async_copy(rows[b], out_hbm.at[pl.ds(o, CH)], ssems[b])
            return carry

        lax.fori_loop(0, ROUNDS, rnd, 0)
        for b in range(NBUF):
            pltpu.make_async_copy(rows[b], out_hbm.at[pl.ds(0, CH)], ssems[b]).wait()

        @pl.when(wid < XW)
        def _extra():
            o = pl.multiple_of((NW * CPW + wid) * CH, 8)
            pltpu.sync_copy(idx_hbm.at[pl.ds(o, CH)], idxs[0])
            pltpu.async_copy(pn_hbm.at[idxs[0]], rows[0], gsems[0]).wait()
            pltpu.sync_copy(rows[0], out_hbm.at[pl.ds(o, CH)])

    return k(pn, idx)


def _sc_scatter(ue, ridx, zrows):
    @functools.partial(
        pl.kernel,
        out_type=jax.ShapeDtypeStruct((NC * NP, H), jnp.float32),
        mesh=_mesh(),
        scratch_types=[
            [pltpu.VMEM((CH,), jnp.int32) for _ in range(NBUF_S)],
            [pltpu.VMEM((CH, H), jnp.float32) for _ in range(NBUF_S)],
            pltpu.VMEM_SHARED((NP, H), jnp.float32),
            [pltpu.SemaphoreType.DMA for _ in range(NBUF_S)],
            [pltpu.SemaphoreType.DMA for _ in range(NBUF_S)],
            [pltpu.SemaphoreType.DMA for _ in range(NBUF_S)],
        ],
    )
    def k(ue_hbm, ridx_hbm, z_hbm, out_hbm, idxs, bufs, agg_sh, isems, dsems, asems):
        cid = lax.axis_index("c")
        sid = lax.axis_index("s")
        wid = sid * NC + cid
        base = wid * CPW
        r0 = sid * RPT
        pltpu.sync_copy(z_hbm.at[pl.ds(r0, RPT)], agg_sh.at[pl.ds(r0, RPT)])
        plsc.subcore_barrier()

        def rnd(g, carry):
            ih, dh = [], []
            for b in range(NBUF_S):
                o = pl.multiple_of((base + g * NBUF_S + b) * CH, 8)

                @pl.when(g > 0)
                def _drain(b=b):
                    pltpu.make_async_copy(bufs[b], agg_sh.at[idxs[b]], asems[b]).wait()

                ih.append(pltpu.async_copy(ridx_hbm.at[pl.ds(o, CH)], idxs[b], isems[b]))
                dh.append(pltpu.async_copy(ue_hbm.at[pl.ds(o, CH)], bufs[b], dsems[b]))
            for b in range(NBUF_S):
                ih[b].wait()
                dh[b].wait()
                pltpu.async_copy(bufs[b], agg_sh.at[idxs[b]], asems[b], add=True)
            return carry

        lax.fori_loop(0, ROUNDS_S, rnd, 0)
        for b in range(NBUF_S):
            pltpu.make_async_copy(bufs[b], agg_sh.at[idxs[b]], asems[b]).wait()

        @pl.when(wid < XW)
        def _extra():
            o = pl.multiple_of((NW * CPW + wid) * CH, 8)
            pltpu.sync_copy(ridx_hbm.at[pl.ds(o, CH)], idxs[0])
            pltpu.sync_copy(ue_hbm.at[pl.ds(o, CH)], bufs[0])
            pltpu.sync_copy(bufs[0], agg_sh.at[idxs[0]], add=True)

        plsc.subcore_barrier()
        pltpu.sync_copy(
            agg_sh.at[pl.ds(r0, RPT)], out_hbm.at[pl.ds(cid * NP + r0, RPT)]
        )

    return k(ue, ridx, zrows)


def _tc_matmul(x, wT):
    R = 2000
    K = x.shape[1]

    def body(x_ref, w_ref, o_ref):
        o_ref[...] = jnp.dot(x_ref[...], w_ref[...], preferred_element_type=jnp.float32)

    return pl.pallas_call(
        body,
        grid=(N // R,),
        in_specs=[
            pl.BlockSpec((R, K), lambda i: (i, 0)),
            pl.BlockSpec((K, H), lambda i: (0, 0)),
        ],
        out_specs=pl.BlockSpec((R, H), lambda i: (i, 0)),
        out_shape=jax.ShapeDtypeStruct((N, H), jnp.float32),
    )(x, wT)


def _tc_edge(x, wT, sg, a):
    R = 8000
    K = x.shape[1]

    def body(x_ref, w_ref, sg_ref, a_ref, o_ref):
        acc = jnp.dot(x_ref[...], w_ref[...], preferred_element_type=jnp.float32)
        acc = acc + sg_ref[...]
        o_ref[...] = jnp.where(acc >= 0, acc, acc * a_ref[...])

    return pl.pallas_call(
        body,
        grid=(E // R,),
        in_specs=[
            pl.BlockSpec((R, K), lambda i: (i, 0)),
            pl.BlockSpec((K, H), lambda i: (0, 0)),
            pl.BlockSpec((R, H), lambda i: (i, 0)),
            pl.BlockSpec((1, H), lambda i: (0, 0)),
        ],
        out_specs=pl.BlockSpec((R, H), lambda i: (i, 0)),
        out_shape=jax.ShapeDtypeStruct((E, H), jnp.float32),
    )(x, wT, sg, a)


def _tc_node(agg0, agg1, x, nwaT, nwnT, na, ewnT):
    R = 2000
    K = x.shape[1]

    def body(a0, a1, x_ref, wa, wn, na_ref, ewn, n_out, pn_out):
        h = jnp.dot(a0[...] + a1[...], wa[...], preferred_element_type=jnp.float32)
        h = h + jnp.dot(x_ref[...], wn[...], preferred_element_type=jnp.float32)
        nn = jnp.where(h >= 0, h, h * na_ref[...])
        n_out[...] = nn
        pn_out[...] = jnp.dot(nn, ewn[...], preferred_element_type=jnp.float32)

    return pl.pallas_call(
        body,
        grid=(N // R,),
        in_specs=[
            pl.BlockSpec((R, H), lambda i: (i, 0)),
            pl.BlockSpec((R, H), lambda i: (i, 0)),
            pl.BlockSpec((R, K), lambda i: (i, 0)),
            pl.BlockSpec((H, H), lambda i: (0, 0)),
            pl.BlockSpec((K, H), lambda i: (0, 0)),
            pl.BlockSpec((1, H), lambda i: (0, 0)),
            pl.BlockSpec((H, H), lambda i: (0, 0)),
        ],
        out_specs=[
            pl.BlockSpec((R, H), lambda i: (i, 0)),
            pl.BlockSpec((R, H), lambda i: (i, 0)),
        ],
        out_shape=[
            jax.ShapeDtypeStruct((N, H), jnp.float32),
            jax.ShapeDtypeStruct((N, H), jnp.float32),
        ],
    )(agg0, agg1, x, nwaT, nwnT, na, ewnT)


def _tc_final(agg0, agg1, x, nwaT, nwnT, na, batch3, fcw, fcb, fca, owT, ob):
    R = 2000
    G = N // R

    def body(a0, a1, x_ref, wa, wn, na_ref, b_ref,
             w1, b1, p1, w2, b2, p2, w3, b3, p3, ow, ob_ref, out_ref, acc):
        j = pl.program_id(0)

        @pl.when(j == 0)
        def _init():
            acc[...] = jnp.zeros_like(acc)

        h = jnp.dot(a0[...] + a1[...], wa[...], preferred_element_type=jnp.float32)
        h = h + jnp.dot(x_ref[...], wn[...], preferred_element_type=jnp.float32)
        nn = jnp.where(h >= 0, h, h * na_ref[...])
        seg = b_ref[0]
        ohT = (lax.broadcasted_iota(jnp.int32, (NG, R), 0) == seg).astype(jnp.float32)
        acc[...] += jnp.dot(ohT, nn, preferred_element_type=jnp.float32)

        @pl.when(j == G - 1)
        def _head():
            u = acc[...]
            for w, b, p in ((w1, b1, p1), (w2, b2, p2), (w3, b3, p3)):
                v = jnp.dot(u, w[...], preferred_element_type=jnp.float32) + b[...]
                u = jnp.where(v >= 0, v, v * p[...])
            out_ref[...] = jnp.dot(u, ow[...], preferred_element_type=jnp.float32) + ob_ref[...]

    wspec = pl.BlockSpec((H, H), lambda i: (0, 0))
    vspec = pl.BlockSpec((1, H), lambda i: (0, 0))
    return pl.pallas_call(
        body,
        grid=(G,),
        in_specs=[
            pl.BlockSpec((R, H), lambda i: (i, 0)),
            pl.BlockSpec((R, H), lambda i: (i, 0)),
            pl.BlockSpec((R, H), lambda i: (i, 0)),
            wspec,
            wspec,
            vspec,
            pl.BlockSpec((1, 1, R), lambda i: (i, 0, 0)),
            wspec, vspec, vspec,
            wspec, vspec, vspec,
            wspec, vspec, vspec,
            pl.BlockSpec((H, 1), lambda i: (0, 0)),
            pl.BlockSpec((1, 1), lambda i: (0, 0)),
        ],
        out_specs=pl.BlockSpec((NG, 1), lambda i: (0, 0)),
        out_shape=jax.ShapeDtypeStruct((NG, 1), jnp.float32),
        scratch_shapes=[pltpu.VMEM((NG, H), jnp.float32)],
    )(agg0, agg1, x, nwaT, nwnT, na, batch3,
      fcw[0], fcb[0], fca[0], fcw[1], fcb[1], fca[1], fcw[2], fcb[2], fca[2],
      owT, ob)


def kernel(n, e, e_i, batch, params):
    recv = e_i[0]
    send = e_i[1]
    gl = params["gl"]
    fc = params["fc"]

    nd0 = n.shape[1]
    n48 = jnp.pad(n, ((0, 0), (0, 48 - nd0)))
    e16 = jnp.pad(e, ((0, 0), (0, 16 - e.shape[1])))
    ewn0T = jnp.pad(gl[0]["ew"][:, :nd0].T, ((0, 48 - nd0), (0, 0)))
    ewe0T = jnp.pad(gl[0]["ew"][:, nd0:].T, ((0, 16 - e.shape[1]), (0, 0)))
    nwa0T = gl[0]["nw"][:, :H].T
    nwn0T = jnp.pad(gl[0]["nw"][:, H:].T, ((0, 48 - nd0), (0, 0)))
    ewn1T, ewe1T = gl[1]["ew"][:, :H].T, gl[1]["ew"][:, H:].T
    nwa1T, nwn1T = gl[1]["nw"][:, :H].T, gl[1]["nw"][:, H:].T
    ewn2T, ewe2T = gl[2]["ew"][:, :H].T, gl[2]["ew"][:, H:].T
    nwa2T, nwn2T = gl[2]["nw"][:, :H].T, gl[2]["nw"][:, H:].T
    ea = [lp["ea"].reshape(1, H) for lp in gl]
    na = [lp["na"].reshape(1, H) for lp in gl]
    fcw = [lp["w"].T for lp in fc]
    fcb = [lp["b"].reshape(1, H) for lp in fc]
    fca = [lp["a"].reshape(1, H) for lp in fc]
    owT = params["ow"].T
    ob = params["ob"].reshape(1, 1)
    batch3 = batch.reshape(N // 2000, 1, 2000)
    zrows = jnp.zeros((NP, H), jnp.float32)

    pn0 = _tc_matmul(n48, ewn0T)
    sg0 = _sc_gather(pn0, send)
    ue0 = _tc_edge(e16, ewe0T, sg0, ea[0])
    ag0 = _sc_scatter(ue0, recv, zrows)
    n1, pn1 = _tc_node(ag0[:N], ag0[NP:NP + N], n48, nwa0T, nwn0T, na[0], ewn1T)
    sg1 = _sc_gather(pn1, send)
    ue1 = _tc_edge(ue0, ewe1T, sg1, ea[1])
    ag1 = _sc_scatter(ue1, recv, zrows)
    n2, pn2 = _tc_node(ag1[:N], ag1[NP:NP + N], n1, nwa1T, nwn1T, na[1], ewn2T)
    sg2 = _sc_gather(pn2, send)
    ue2 = _tc_edge(ue1, ewe2T, sg2, ea[2])
    ag2 = _sc_scatter(ue2, recv, zrows)
    return _tc_final(ag2[:N], ag2[NP:NP + N], n2, nwa2T, nwn2T, na[2], batch3,
                     fcw, fcb, fca, owT, ob)
